# Initial kernel scaffold; baseline (speedup 1.0000x reference)
import functools

import jax
import jax.numpy as jnp
from jax import lax
from jax.experimental import pallas as pl
from jax.experimental.pallas import tpu as pltpu
from jax.experimental.pallas import tpu_sc as plsc

N = 10000
E = 320000
H = 128
L = 3
G = 64
NUM_ATOM = 119
NUM_EDGE = 5

NW = 32
CHUNK = 128
CPT = 80
E_PAD = NW * CPT * CHUNK
N_PAD = 10240
ROWS_PER_TILE = N_PAD // 16
NB = 10
BN_ = N // NB
HIGH = lax.Precision.HIGHEST


def _h0_body(x_ref, emb_ref, o_ref):
    iota = lax.broadcasted_iota(jnp.float32, (BN_, 128), 1)
    onehot = (x_ref[...] == iota).astype(jnp.float32)
    o_ref[...] = jnp.dot(onehot, emb_ref[...],
                         preferred_element_type=jnp.float32, precision=HIGH)


def _h0(x_f, emb_pad):
    return pl.pallas_call(
        _h0_body,
        grid=(NB,),
        in_specs=[pl.BlockSpec((BN_, 1), lambda i: (i, 0)),
                  pl.BlockSpec((128, 128), lambda i: (0, 0))],
        out_specs=pl.BlockSpec((BN_, 128), lambda i: (i, 0)),
        out_shape=jax.ShapeDtypeStruct((N, H), jnp.float32),
    )(x_f, emb_pad)


def _gidx_body(src_ref, attr_ref, o_ref):
    o_ref[...] = attr_ref[...] * N + src_ref[...]


def _gidx(src_r, attr_r):
    return pl.pallas_call(
        _gidx_body,
        grid=(NW,),
        in_specs=[pl.BlockSpec((1, CPT, CHUNK), lambda i: (i, 0, 0)),
                  pl.BlockSpec((1, CPT, CHUNK), lambda i: (i, 0, 0))],
        out_specs=pl.BlockSpec((1, CPT, CHUNK), lambda i: (i, 0, 0)),
        out_shape=jax.ShapeDtypeStruct((NW, CPT, CHUNK), jnp.int32),
    )(src_r, attr_r)


def _comb_body(h_ref, batch_ref, vn_ref, bond_ref, hl_ref, comb_ref):
    iota = lax.broadcasted_iota(jnp.float32, (BN_, G), 1)
    onehot = (batch_ref[...] == iota).astype(jnp.float32)
    vnb = jnp.dot(onehot, vn_ref[...],
                  preferred_element_type=jnp.float32, precision=HIGH)
    hl = h_ref[...] + vnb
    hl_ref[...] = hl
    for a in range(NUM_EDGE):
        comb_ref[a] = jnp.maximum(hl + bond_ref[a], 0.0)


def _comb(h, batch_f, vn, bond):
    return pl.pallas_call(
        _comb_body,
        grid=(NB,),
        in_specs=[pl.BlockSpec((BN_, 128), lambda i: (i, 0)),
                  pl.BlockSpec((BN_, 1), lambda i: (i, 0)),
                  pl.BlockSpec((G, 128), lambda i: (0, 0)),
                  pl.BlockSpec((NUM_EDGE, 128), lambda i: (0, 0))],
        out_specs=[pl.BlockSpec((BN_, 128), lambda i: (i, 0)),
                   pl.BlockSpec((NUM_EDGE, BN_, 128), lambda i: (0, i, 0))],
        out_shape=[jax.ShapeDtypeStruct((N, H), jnp.float32),
                   jax.ShapeDtypeStruct((NUM_EDGE, N, H), jnp.float32)],
    )(h, batch_f, vn, bond)


def _bn(t, scale, bias):
    m = jnp.mean(t, axis=0)
    d = t - m
    v = jnp.mean(d * d, axis=0)
    return scale * d / jnp.sqrt(v + 1e-5) + bias


def _mlp_body(last, hl_ref, agg_ref, hin_ref, eps_ref, w1_ref, b1_ref,
              s1_ref, c1_ref, w2_ref, b2_ref, s2_ref, c2_ref, batch_ref,
              out_ref, hg_ref=None):
    agg = agg_ref[0, :N, :] + agg_ref[1, :N, :]
    z0 = (1.0 + eps_ref[0, 0]) * hl_ref[...] + agg
    t = jnp.dot(z0, w1_ref[...],
                preferred_element_type=jnp.float32, precision=HIGH) + b1_ref[...]
    t = jnp.maximum(_bn(t, s1_ref[...], c1_ref[...]), 0.0)
    u = jnp.dot(t, w2_ref[...],
                preferred_element_type=jnp.float32, precision=HIGH) + b2_ref[...]
    u = _bn(u, s2_ref[...], c2_ref[...])
    if not last:
        u = jnp.maximum(u, 0.0)
    out = u + hin_ref[...]
    out_ref[...] = out
    if last:
        iota = lax.broadcasted_iota(jnp.float32, (G, N), 0)
        onehot_t = (batch_ref[...] == iota).astype(jnp.float32)
        hg_ref[...] = jnp.dot(onehot_t, out,
                              preferred_element_type=jnp.float32, precision=HIGH)


def _mlp(hl, agg2, h_in, eps, p, batch_row, last):
    full = lambda s: pl.BlockSpec(s, lambda: tuple(0 for _ in s))
    out_shape = [jax.ShapeDtypeStruct((N, H), jnp.float32)]
    out_specs = [full((N, 128))]
    if last:
        out_shape.append(jax.ShapeDtypeStruct((G, H), jnp.float32))
        out_specs.append(full((G, 128)))
    r = pl.pallas_call(
        functools.partial(_mlp_body, last),
        in_specs=[full((N, 128)), full((2, N_PAD, 128)), full((N, 128)),
                  full((1, 1)), full((128, 256)), full((256,)), full((256,)),
                  full((256,)), full((256, 128)), full((128,)), full((128,)),
                  full((128,)), full((1, N))],
        out_specs=out_specs,
        out_shape=out_shape,
    )(hl, agg2, h_in, eps, p['W1'], p['b1'], p['mlp_bn_scale'],
      p['mlp_bn_bias'], p['W2'], p['b2'], p['bn_scale'], p['bn_bias'],
      batch_row)
    return r if last else (r[0], None)


def _vn_body(h_ref, batch_ref, vn_ref, w1_ref, b1_ref, s1_ref, c1_ref,
             w2_ref, b2_ref, s2_ref, c2_ref, out_ref):
    iota = lax.broadcasted_iota(jnp.float32, (G, N), 0)
    onehot_t = (batch_ref[...] == iota).astype(jnp.float32)
    seg = jnp.dot(onehot_t, h_ref[...],
                  preferred_element_type=jnp.float32, precision=HIGH)
    vt = seg + vn_ref[...]
    t = jnp.dot(vt, w1_ref[...],
                preferred_element_type=jnp.float32, precision=HIGH) + b1_ref[...]
    t = jnp.maximum(_bn(t, s1_ref[...], c1_ref[...]), 0.0)
    u = jnp.dot(t, w2_ref[...],
                preferred_element_type=jnp.float32, precision=HIGH) + b2_ref[...]
    u = _bn(u, s2_ref[...], c2_ref[...])
    out_ref[...] = jnp.maximum(u, 0.0)


def _vn_mlp(h, batch_row, vn, q):
    full = lambda s: pl.BlockSpec(s, lambda: tuple(0 for _ in s))
    return pl.pallas_call(
        _vn_body,
        in_specs=[full((N, 128)), full((1, N)), full((G, 128)),
                  full((128, 256)), full((256,)), full((256,)), full((256,)),
                  full((256, 128)), full((128,)), full((128,)), full((128,))],
        out_specs=full((G, 128)),
        out_shape=jax.ShapeDtypeStruct((G, H), jnp.float32),
    )(h, batch_row, vn, q['W1'], q['b1'], q['bn1_scale'], q['bn1_bias'],
      q['W2'], q['b2'], q['bn2_scale'], q['bn2_bias'])


_SC_MESH = plsc.VectorSubcoreMesh(core_axis_name="c", subcore_axis_name="s")


@functools.partial(
    pl.kernel,
    out_type=jax.ShapeDtypeStruct((2, N_PAD, H), jnp.float32),
    mesh=_SC_MESH,
    scratch_types=[
        pltpu.VMEM((CPT, CHUNK), jnp.int32),
        pltpu.VMEM((CPT, CHUNK), jnp.int32),
        pltpu.VMEM((CHUNK, H), jnp.float32),
        pltpu.VMEM_SHARED((N_PAD, H), jnp.float32),
        pltpu.SemaphoreType.DMA,
    ],
)
def _sc_edge(comb_hbm, gidx_hbm, didx_hbm, zeros_hbm, out_hbm,
             gidx_v, didx_v, rows_v, agg_sh, sem):
    c = lax.axis_index("c")
    s = lax.axis_index("s")
    wid = s * 2 + c

    pltpu.sync_copy(gidx_hbm.at[wid], gidx_v)
    pltpu.sync_copy(didx_hbm.at[wid], didx_v)

    pltpu.sync_copy(zeros_hbm, agg_sh.at[pl.ds(s * ROWS_PER_TILE, ROWS_PER_TILE)])
    plsc.subcore_barrier()

    @pl.loop(0, CPT)
    def _(j):
        pltpu.async_copy(comb_hbm.at[gidx_v.at[j]], rows_v, sem).wait()
        pltpu.sync_copy(rows_v, agg_sh.at[didx_v.at[j]], add=True)

    plsc.subcore_barrier()
    pltpu.sync_copy(agg_sh.at[pl.ds(s * ROWS_PER_TILE, ROWS_PER_TILE)],
                    out_hbm.at[c, pl.ds(s * ROWS_PER_TILE, ROWS_PER_TILE)])


def kernel(x, edge_index, edge_attr, batch, params):
    x_f = x.astype(jnp.float32).reshape(N, 1)
    batch_f = batch.astype(jnp.float32).reshape(N, 1)
    batch_row = batch.astype(jnp.float32).reshape(1, N)

    src = edge_index[0].astype(jnp.int32)
    dst = edge_index[1].astype(jnp.int32)
    attr = edge_attr.astype(jnp.int32)
    npad = E_PAD - E
    src_r = jnp.concatenate([src, jnp.zeros((npad,), jnp.int32)]
                            ).reshape(NW, CPT, CHUNK)
    attr_r = jnp.concatenate([attr, jnp.zeros((npad,), jnp.int32)]
                             ).reshape(NW, CPT, CHUNK)
    didx = jnp.concatenate([dst, jnp.full((npad,), N_PAD - 8, jnp.int32)]
                           ).reshape(NW, CPT, CHUNK)
    gidx = _gidx(src_r, attr_r)

    emb_pad = jnp.zeros((128, 128), jnp.float32).at[:NUM_ATOM].set(
        params['atom_emb'])
    zeros_hbm = jnp.zeros((ROWS_PER_TILE, H), jnp.float32)

    h = _h0(x_f, emb_pad)
    vn = jnp.broadcast_to(params['vn_emb'], (G, H))

    hg = None
    for l in range(L):
        p = params['layers'][l]
        hl, comb = _comb(h, batch_f, vn, p['bond_emb'])
        agg2 = _sc_edge(comb.reshape(NUM_EDGE * N, H), gidx, didx, zeros_hbm)
        if l < L - 1:
            vn = _vn_mlp(h, batch_row, vn, params['vn_mlps'][l])
        eps = p['eps'].reshape(1, 1)
        h, hg = _mlp(hl, agg2, h, eps, p, batch_row, last=(l == L - 1))

    return hg, h


# speedup vs baseline: 3.0120x; 3.0120x over previous
import functools

import jax
import jax.numpy as jnp
from jax import lax
from jax.experimental import pallas as pl
from jax.experimental.pallas import tpu as pltpu
from jax.experimental.pallas import tpu_sc as plsc

N = 10000
E = 320000
H = 128
L = 3
G = 64
NUM_ATOM = 119
NUM_EDGE = 5

NW = 32
CHUNK = 128
CPT = 80
E_PAD = NW * CPT * CHUNK
N_PAD = 10240
ROWS_PER_TILE = N_PAD // 16
NB = 10
BN_ = N // NB
HIGH = lax.Precision.HIGHEST


def _h0_body(x_ref, emb_ref, o_ref):
    iota = lax.broadcasted_iota(jnp.int32, (BN_, 128), 1).astype(jnp.float32)
    onehot = (x_ref[...] == iota).astype(jnp.float32)
    o_ref[...] = jnp.dot(onehot, emb_ref[...],
                         preferred_element_type=jnp.float32, precision=HIGH)


def _h0(x_f, emb_pad):
    return pl.pallas_call(
        _h0_body,
        grid=(NB,),
        in_specs=[pl.BlockSpec((BN_, 1), lambda i: (i, 0)),
                  pl.BlockSpec((128, 128), lambda i: (0, 0))],
        out_specs=pl.BlockSpec((BN_, 128), lambda i: (i, 0)),
        out_shape=jax.ShapeDtypeStruct((N, H), jnp.float32),
    )(x_f, emb_pad)


def _gidx_body(src_ref, attr_ref, o_ref):
    o_ref[...] = attr_ref[...] * N + src_ref[...]


def _gidx(src_r, attr_r):
    return pl.pallas_call(
        _gidx_body,
        grid=(NW,),
        in_specs=[pl.BlockSpec((1, CPT, CHUNK), lambda i: (i, 0, 0)),
                  pl.BlockSpec((1, CPT, CHUNK), lambda i: (i, 0, 0))],
        out_specs=pl.BlockSpec((1, CPT, CHUNK), lambda i: (i, 0, 0)),
        out_shape=jax.ShapeDtypeStruct((NW, CPT, CHUNK), jnp.int32),
    )(src_r, attr_r)


def _comb_body(h_ref, batch_ref, vn_ref, bond_ref, hl_ref, comb_ref):
    iota = lax.broadcasted_iota(jnp.int32, (BN_, G), 1).astype(jnp.float32)
    onehot = (batch_ref[...] == iota).astype(jnp.float32)
    vnb = jnp.dot(onehot, vn_ref[...],
                  preferred_element_type=jnp.float32, precision=HIGH)
    hl = h_ref[...] + vnb
    hl_ref[...] = hl
    for a in range(NUM_EDGE):
        comb_ref[a] = jnp.maximum(hl + bond_ref[a], 0.0)


def _comb(h, batch_f, vn, bond):
    return pl.pallas_call(
        _comb_body,
        grid=(NB,),
        in_specs=[pl.BlockSpec((BN_, 128), lambda i: (i, 0)),
                  pl.BlockSpec((BN_, 1), lambda i: (i, 0)),
                  pl.BlockSpec((G, 128), lambda i: (0, 0)),
                  pl.BlockSpec((NUM_EDGE, 128), lambda i: (0, 0))],
        out_specs=[pl.BlockSpec((BN_, 128), lambda i: (i, 0)),
                   pl.BlockSpec((NUM_EDGE, BN_, 128), lambda i: (0, i, 0))],
        out_shape=[jax.ShapeDtypeStruct((N, H), jnp.float32),
                   jax.ShapeDtypeStruct((NUM_EDGE, N, H), jnp.float32)],
    )(h, batch_f, vn, bond)


def _bn(t, scale, bias):
    m = jnp.mean(t, axis=0)
    d = t - m
    v = jnp.mean(d * d, axis=0)
    return scale * d / jnp.sqrt(v + 1e-5) + bias


def _mlp_body(last, hl_ref, agg_ref, hin_ref, eps_ref, w1_ref, b1_ref,
              s1_ref, c1_ref, w2_ref, b2_ref, s2_ref, c2_ref, batch_ref,
              out_ref, hg_ref=None):
    agg = agg_ref[0, :N, :] + agg_ref[1, :N, :]
    z0 = (1.0 + eps_ref[0, 0]) * hl_ref[...] + agg
    t = jnp.dot(z0, w1_ref[...],
                preferred_element_type=jnp.float32) + b1_ref[...]
    t = jnp.maximum(_bn(t, s1_ref[...], c1_ref[...]), 0.0)
    u = jnp.dot(t, w2_ref[...],
                preferred_element_type=jnp.float32) + b2_ref[...]
    u = _bn(u, s2_ref[...], c2_ref[...])
    if not last:
        u = jnp.maximum(u, 0.0)
    out = u + hin_ref[...]
    out_ref[...] = out
    if last:
        iota = lax.broadcasted_iota(jnp.int32, (G, N), 0).astype(jnp.float32)
        onehot_t = (batch_ref[...] == iota).astype(jnp.float32)
        hg_ref[...] = jnp.dot(onehot_t, out,
                              preferred_element_type=jnp.float32, precision=HIGH)


def _mlp(hl, agg2, h_in, eps, p, batch_row, last):
    full = lambda s: pl.BlockSpec(s, lambda: tuple(0 for _ in s))
    out_shape = [jax.ShapeDtypeStruct((N, H), jnp.float32)]
    out_specs = [full((N, 128))]
    if last:
        out_shape.append(jax.ShapeDtypeStruct((G, H), jnp.float32))
        out_specs.append(full((G, 128)))
    r = pl.pallas_call(
        functools.partial(_mlp_body, last),
        in_specs=[full((N, 128)), full((2, N_PAD, 128)), full((N, 128)),
                  full((1, 1)), full((128, 256)), full((256,)), full((256,)),
                  full((256,)), full((256, 128)), full((128,)), full((128,)),
                  full((128,)), full((1, N))],
        out_specs=out_specs,
        out_shape=out_shape,
    )(hl, agg2, h_in, eps, p['W1'], p['b1'], p['mlp_bn_scale'],
      p['mlp_bn_bias'], p['W2'], p['b2'], p['bn_scale'], p['bn_bias'],
      batch_row)
    return r if last else (r[0], None)


def _vn_body(h_ref, batch_ref, vn_ref, w1_ref, b1_ref, s1_ref, c1_ref,
             w2_ref, b2_ref, s2_ref, c2_ref, out_ref):
    iota = lax.broadcasted_iota(jnp.int32, (G, N), 0).astype(jnp.float32)
    onehot_t = (batch_ref[...] == iota).astype(jnp.float32)
    seg = jnp.dot(onehot_t, h_ref[...],
                  preferred_element_type=jnp.float32, precision=HIGH)
    vt = seg + vn_ref[...]
    t = jnp.dot(vt, w1_ref[...],
                preferred_element_type=jnp.float32) + b1_ref[...]
    t = jnp.maximum(_bn(t, s1_ref[...], c1_ref[...]), 0.0)
    u = jnp.dot(t, w2_ref[...],
                preferred_element_type=jnp.float32) + b2_ref[...]
    u = _bn(u, s2_ref[...], c2_ref[...])
    out_ref[...] = jnp.maximum(u, 0.0)


def _vn_mlp(h, batch_row, vn, q):
    full = lambda s: pl.BlockSpec(s, lambda: tuple(0 for _ in s))
    return pl.pallas_call(
        _vn_body,
        in_specs=[full((N, 128)), full((1, N)), full((G, 128)),
                  full((128, 256)), full((256,)), full((256,)), full((256,)),
                  full((256, 128)), full((128,)), full((128,)), full((128,))],
        out_specs=full((G, 128)),
        out_shape=jax.ShapeDtypeStruct((G, H), jnp.float32),
    )(h, batch_row, vn, q['W1'], q['b1'], q['bn1_scale'], q['bn1_bias'],
      q['W2'], q['b2'], q['bn2_scale'], q['bn2_bias'])


@functools.cache
def _get_sc_edge():
    mesh = plsc.VectorSubcoreMesh(core_axis_name="c", subcore_axis_name="s")

    @functools.partial(
        pl.kernel,
        out_type=jax.ShapeDtypeStruct((2, N_PAD, H), jnp.float32),
        mesh=mesh,
        scratch_types=[
            pltpu.VMEM((CPT, CHUNK), jnp.int32),
            pltpu.VMEM((CPT, CHUNK), jnp.int32),
            pltpu.VMEM((CHUNK, H), jnp.float32),
            pltpu.VMEM_SHARED((N_PAD, H), jnp.float32),
            pltpu.SemaphoreType.DMA,
        ],
    )
    def _sc_edge_kernel(comb_hbm, gidx_hbm, didx_hbm, zeros_hbm, out_hbm,
                        gidx_v, didx_v, rows_v, agg_sh, sem):
        c = lax.axis_index("c")
        s = lax.axis_index("s")
        wid = s * 2 + c

        pltpu.sync_copy(gidx_hbm.at[wid], gidx_v)
        pltpu.sync_copy(didx_hbm.at[wid], didx_v)

        pltpu.sync_copy(zeros_hbm,
                        agg_sh.at[pl.ds(s * ROWS_PER_TILE, ROWS_PER_TILE)])
        plsc.subcore_barrier()

        @pl.loop(0, CPT)
        def _(j):
            pltpu.async_copy(comb_hbm.at[gidx_v.at[j]], rows_v, sem).wait()
            pltpu.sync_copy(rows_v, agg_sh.at[didx_v.at[j]], add=True)

        plsc.subcore_barrier()
        pltpu.sync_copy(agg_sh.at[pl.ds(s * ROWS_PER_TILE, ROWS_PER_TILE)],
                        out_hbm.at[c, pl.ds(s * ROWS_PER_TILE, ROWS_PER_TILE)])

    return _sc_edge_kernel


def _sc_edge(comb, gidx, didx, zeros_hbm):
    return _get_sc_edge()(comb, gidx, didx, zeros_hbm)


def kernel(x, edge_index, edge_attr, batch, params):
    x_f = x.astype(jnp.float32).reshape(N, 1)
    batch_f = batch.astype(jnp.float32).reshape(N, 1)
    batch_row = batch.astype(jnp.float32).reshape(1, N)

    src = edge_index[0].astype(jnp.int32)
    dst = edge_index[1].astype(jnp.int32)
    attr = edge_attr.astype(jnp.int32)
    npad = E_PAD - E
    src_r = jnp.concatenate([src, jnp.zeros((npad,), jnp.int32)]
                            ).reshape(NW, CPT, CHUNK)
    attr_r = jnp.concatenate([attr, jnp.zeros((npad,), jnp.int32)]
                             ).reshape(NW, CPT, CHUNK)
    didx = jnp.concatenate([dst, jnp.full((npad,), N_PAD - 8, jnp.int32)]
                           ).reshape(NW, CPT, CHUNK)
    gidx = _gidx(src_r, attr_r)

    emb_pad = jnp.zeros((128, 128), jnp.float32).at[:NUM_ATOM].set(
        params['atom_emb'])
    zeros_hbm = jnp.zeros((ROWS_PER_TILE, H), jnp.float32)

    h = _h0(x_f, emb_pad)
    vn = jnp.broadcast_to(params['vn_emb'], (G, H))

    hg = None
    for l in range(L):
        p = params['layers'][l]
        hl, comb = _comb(h, batch_f, vn, p['bond_emb'])
        agg2 = _sc_edge(comb.reshape(NUM_EDGE * N, H), gidx, didx, zeros_hbm)
        if l < L - 1:
            vn = _vn_mlp(h, batch_row, vn, params['vn_mlps'][l])
        eps = p['eps'].reshape(1, 1)
        h, hg = _mlp(hl, agg2, h, eps, p, batch_row, last=(l == L - 1))

    return hg, h


# speedup vs baseline: 3.2732x; 1.0867x over previous
import functools

import jax
import jax.numpy as jnp
from jax import lax
from jax.experimental import pallas as pl
from jax.experimental.pallas import tpu as pltpu
from jax.experimental.pallas import tpu_sc as plsc

N = 10000
E = 320000
H = 128
L = 3
G = 64
NUM_ATOM = 119
NUM_EDGE = 5

NW = 32
CHUNK = 128
CPT = 80
E_PAD = NW * CPT * CHUNK
N_PAD = 10240
ROWS_PER_TILE = N_PAD // 16
NB = 10
BN_ = N // NB
HIGH = lax.Precision.HIGHEST


def _h0_body(x_ref, emb_ref, o_ref):
    iota = lax.broadcasted_iota(jnp.int32, (BN_, 128), 1).astype(jnp.float32)
    onehot = (x_ref[...] == iota).astype(jnp.float32)
    o_ref[...] = jnp.dot(onehot, emb_ref[...],
                         preferred_element_type=jnp.float32, precision=HIGH)


def _h0(x_f, emb_pad):
    return pl.pallas_call(
        _h0_body,
        grid=(NB,),
        in_specs=[pl.BlockSpec((BN_, 1), lambda i: (i, 0)),
                  pl.BlockSpec((128, 128), lambda i: (0, 0))],
        out_specs=pl.BlockSpec((BN_, 128), lambda i: (i, 0)),
        out_shape=jax.ShapeDtypeStruct((N, H), jnp.float32),
    )(x_f, emb_pad)


def _gidx_body(src_ref, attr_ref, o_ref):
    o_ref[...] = attr_ref[...] * N + src_ref[...]


def _gidx(src_r, attr_r):
    return pl.pallas_call(
        _gidx_body,
        grid=(NW,),
        in_specs=[pl.BlockSpec((1, CPT, CHUNK), lambda i: (i, 0, 0)),
                  pl.BlockSpec((1, CPT, CHUNK), lambda i: (i, 0, 0))],
        out_specs=pl.BlockSpec((1, CPT, CHUNK), lambda i: (i, 0, 0)),
        out_shape=jax.ShapeDtypeStruct((NW, CPT, CHUNK), jnp.int32),
    )(src_r, attr_r)


def _comb_body(h_ref, batch_ref, vn_ref, bond_ref, hl_ref, comb_ref):
    iota = lax.broadcasted_iota(jnp.int32, (BN_, G), 1).astype(jnp.float32)
    onehot = (batch_ref[...] == iota).astype(jnp.float32)
    vnb = jnp.dot(onehot, vn_ref[...],
                  preferred_element_type=jnp.float32, precision=HIGH)
    hl = h_ref[...] + vnb
    hl_ref[...] = hl
    for a in range(NUM_EDGE):
        comb_ref[a] = jnp.maximum(hl + bond_ref[a], 0.0)


def _comb(h, batch_f, vn, bond):
    return pl.pallas_call(
        _comb_body,
        grid=(NB,),
        in_specs=[pl.BlockSpec((BN_, 128), lambda i: (i, 0)),
                  pl.BlockSpec((BN_, 1), lambda i: (i, 0)),
                  pl.BlockSpec((G, 128), lambda i: (0, 0)),
                  pl.BlockSpec((NUM_EDGE, 128), lambda i: (0, 0))],
        out_specs=[pl.BlockSpec((BN_, 128), lambda i: (i, 0)),
                   pl.BlockSpec((NUM_EDGE, BN_, 128), lambda i: (0, i, 0))],
        out_shape=[jax.ShapeDtypeStruct((N, H), jnp.float32),
                   jax.ShapeDtypeStruct((NUM_EDGE, N, H), jnp.float32)],
    )(h, batch_f, vn, bond)


def _bn(t, scale, bias):
    m = jnp.mean(t, axis=0)
    d = t - m
    v = jnp.mean(d * d, axis=0)
    return scale * d / jnp.sqrt(v + 1e-5) + bias


def _mlp_body(last, hl_ref, agg_ref, hin_ref, eps_ref, w1_ref, b1_ref,
              s1_ref, c1_ref, w2_ref, b2_ref, s2_ref, c2_ref, batch_ref,
              out_ref, hg_ref=None):
    agg = agg_ref[0, :N, :] + agg_ref[1, :N, :]
    z0 = (1.0 + eps_ref[0, 0]) * hl_ref[...] + agg
    t = jnp.dot(z0, w1_ref[...],
                preferred_element_type=jnp.float32) + b1_ref[...]
    t = jnp.maximum(_bn(t, s1_ref[...], c1_ref[...]), 0.0)
    u = jnp.dot(t, w2_ref[...],
                preferred_element_type=jnp.float32) + b2_ref[...]
    u = _bn(u, s2_ref[...], c2_ref[...])
    if not last:
        u = jnp.maximum(u, 0.0)
    out = u + hin_ref[...]
    out_ref[...] = out
    if last:
        iota = lax.broadcasted_iota(jnp.int32, (G, N), 0).astype(jnp.float32)
        onehot_t = (batch_ref[...] == iota).astype(jnp.float32)
        hg_ref[...] = jnp.dot(onehot_t, out,
                              preferred_element_type=jnp.float32, precision=HIGH)


def _mlp(hl, agg2, h_in, eps, p, batch_row, last):
    full = lambda s: pl.BlockSpec(s, lambda: tuple(0 for _ in s))
    out_shape = [jax.ShapeDtypeStruct((N, H), jnp.float32)]
    out_specs = [full((N, 128))]
    if last:
        out_shape.append(jax.ShapeDtypeStruct((G, H), jnp.float32))
        out_specs.append(full((G, 128)))
    r = pl.pallas_call(
        functools.partial(_mlp_body, last),
        in_specs=[full((N, 128)), full((2, N_PAD, 128)), full((N, 128)),
                  full((1, 1)), full((128, 256)), full((256,)), full((256,)),
                  full((256,)), full((256, 128)), full((128,)), full((128,)),
                  full((128,)), full((1, N))],
        out_specs=out_specs,
        out_shape=out_shape,
    )(hl, agg2, h_in, eps, p['W1'], p['b1'], p['mlp_bn_scale'],
      p['mlp_bn_bias'], p['W2'], p['b2'], p['bn_scale'], p['bn_bias'],
      batch_row)
    return r if last else (r[0], None)


def _vn_body(h_ref, batch_ref, vn_ref, w1_ref, b1_ref, s1_ref, c1_ref,
             w2_ref, b2_ref, s2_ref, c2_ref, out_ref):
    iota = lax.broadcasted_iota(jnp.int32, (G, N), 0).astype(jnp.float32)
    onehot_t = (batch_ref[...] == iota).astype(jnp.float32)
    seg = jnp.dot(onehot_t, h_ref[...],
                  preferred_element_type=jnp.float32, precision=HIGH)
    vt = seg + vn_ref[...]
    t = jnp.dot(vt, w1_ref[...],
                preferred_element_type=jnp.float32) + b1_ref[...]
    t = jnp.maximum(_bn(t, s1_ref[...], c1_ref[...]), 0.0)
    u = jnp.dot(t, w2_ref[...],
                preferred_element_type=jnp.float32) + b2_ref[...]
    u = _bn(u, s2_ref[...], c2_ref[...])
    out_ref[...] = jnp.maximum(u, 0.0)


def _vn_mlp(h, batch_row, vn, q):
    full = lambda s: pl.BlockSpec(s, lambda: tuple(0 for _ in s))
    return pl.pallas_call(
        _vn_body,
        in_specs=[full((N, 128)), full((1, N)), full((G, 128)),
                  full((128, 256)), full((256,)), full((256,)), full((256,)),
                  full((256, 128)), full((128,)), full((128,)), full((128,))],
        out_specs=full((G, 128)),
        out_shape=jax.ShapeDtypeStruct((G, H), jnp.float32),
    )(h, batch_row, vn, q['W1'], q['b1'], q['bn1_scale'], q['bn1_bias'],
      q['W2'], q['b2'], q['bn2_scale'], q['bn2_bias'])


@functools.cache
def _get_sc_edge():
    mesh = plsc.VectorSubcoreMesh(core_axis_name="c", subcore_axis_name="s")

    NSLOT = 2
    HCPT = CPT // 2
    NGRP = HCPT // NSLOT

    @functools.partial(
        pl.kernel,
        out_type=jax.ShapeDtypeStruct((2, N_PAD, H), jnp.float32),
        mesh=mesh,
        scratch_types=[
            pltpu.VMEM((HCPT, CHUNK), jnp.int32),
            pltpu.VMEM((HCPT, CHUNK), jnp.int32),
            pltpu.VMEM((NSLOT, CHUNK, H), jnp.float32),
            pltpu.VMEM_SHARED((N_PAD, H), jnp.float32),
            [pltpu.SemaphoreType.DMA] * NSLOT,
            [pltpu.SemaphoreType.DMA] * NSLOT,
        ],
    )
    def _sc_edge_kernel(comb_hbm, gidx_hbm, didx_hbm, zeros_hbm, out_hbm,
                        gidx_v, didx_v, rows_v, agg_sh, gsem, ssem):
        c = lax.axis_index("c")
        s = lax.axis_index("s")
        wid = s * 2 + c

        pltpu.sync_copy(zeros_hbm,
                        agg_sh.at[pl.ds(s * ROWS_PER_TILE, ROWS_PER_TILE)])
        plsc.subcore_barrier()

        def wait_slot(b, sem):
            pltpu.make_async_copy(comb_hbm.at[pl.ds(0, CHUNK)],
                                  rows_v.at[b], sem[b]).wait()

        for half in range(2):
            pltpu.sync_copy(gidx_hbm.at[wid, pl.ds(half * HCPT, HCPT)], gidx_v)
            pltpu.sync_copy(didx_hbm.at[wid, pl.ds(half * HCPT, HCPT)], didx_v)

            for b in range(NSLOT):
                pltpu.async_copy(comb_hbm.at[gidx_v.at[b]], rows_v.at[b],
                                 gsem[b])

            @pl.loop(0, NGRP - 1)
            def _(g):
                for b in range(NSLOT):
                    wait_slot(b, gsem)
                    pltpu.async_copy(rows_v.at[b],
                                     agg_sh.at[didx_v.at[g * NSLOT + b]],
                                     ssem[b], add=True)
                for b in range(NSLOT):
                    wait_slot(b, ssem)
                    pltpu.async_copy(
                        comb_hbm.at[gidx_v.at[(g + 1) * NSLOT + b]],
                        rows_v.at[b], gsem[b])

            for b in range(NSLOT):
                wait_slot(b, gsem)
                pltpu.async_copy(rows_v.at[b],
                                 agg_sh.at[didx_v.at[HCPT - NSLOT + b]],
                                 ssem[b], add=True)
            for b in range(NSLOT):
                wait_slot(b, ssem)

        plsc.subcore_barrier()
        pltpu.sync_copy(agg_sh.at[pl.ds(s * ROWS_PER_TILE, ROWS_PER_TILE)],
                        out_hbm.at[c, pl.ds(s * ROWS_PER_TILE, ROWS_PER_TILE)])

    return _sc_edge_kernel


def _sc_edge(comb, gidx, didx, zeros_hbm):
    return _get_sc_edge()(comb, gidx, didx, zeros_hbm)


def kernel(x, edge_index, edge_attr, batch, params):
    x_f = x.astype(jnp.float32).reshape(N, 1)
    batch_f = batch.astype(jnp.float32).reshape(N, 1)
    batch_row = batch.astype(jnp.float32).reshape(1, N)

    src = edge_index[0].astype(jnp.int32)
    dst = edge_index[1].astype(jnp.int32)
    attr = edge_attr.astype(jnp.int32)
    npad = E_PAD - E
    src_r = jnp.concatenate([src, jnp.zeros((npad,), jnp.int32)]
                            ).reshape(NW, CPT, CHUNK)
    attr_r = jnp.concatenate([attr, jnp.zeros((npad,), jnp.int32)]
                             ).reshape(NW, CPT, CHUNK)
    didx = jnp.concatenate([dst, jnp.full((npad,), N_PAD - 8, jnp.int32)]
                           ).reshape(NW, CPT, CHUNK)
    gidx = _gidx(src_r, attr_r)

    emb_pad = jnp.zeros((128, 128), jnp.float32).at[:NUM_ATOM].set(
        params['atom_emb'])
    zeros_hbm = jnp.zeros((ROWS_PER_TILE, H), jnp.float32)

    h = _h0(x_f, emb_pad)
    vn = jnp.broadcast_to(params['vn_emb'], (G, H))

    hg = None
    for l in range(L):
        p = params['layers'][l]
        hl, comb = _comb(h, batch_f, vn, p['bond_emb'])
        agg2 = _sc_edge(comb.reshape(NUM_EDGE * N, H), gidx, didx, zeros_hbm)
        if l < L - 1:
            vn = _vn_mlp(h, batch_row, vn, params['vn_mlps'][l])
        eps = p['eps'].reshape(1, 1)
        h, hg = _mlp(hl, agg2, h, eps, p, batch_row, last=(l == L - 1))

    return hg, h


# speedup vs baseline: 9.0122x; 2.7533x over previous
import functools

import jax
import jax.numpy as jnp
from jax import lax
from jax.experimental import pallas as pl
from jax.experimental.pallas import tpu as pltpu
from jax.experimental.pallas import tpu_sc as plsc

N = 10000
E = 320000
H = 128
L = 3
G = 64
NUM_ATOM = 119
NUM_EDGE = 5

NW = 32
CHUNK = 128
CPT = 80
E_PAD = NW * CPT * CHUNK
N_PAD = 10240
ROWS_PER_TILE = N_PAD // 16
NB = 10
BN_ = N // NB
HIGH = lax.Precision.HIGHEST


def _h0_body(x_ref, emb_ref, o_ref):
    iota = lax.broadcasted_iota(jnp.int32, (BN_, 128), 1).astype(jnp.float32)
    onehot = (x_ref[...] == iota).astype(jnp.float32)
    o_ref[...] = jnp.dot(onehot, emb_ref[...],
                         preferred_element_type=jnp.float32, precision=HIGH)


def _h0(x_f, emb_pad):
    return pl.pallas_call(
        _h0_body,
        grid=(NB,),
        in_specs=[pl.BlockSpec((BN_, 1), lambda i: (i, 0)),
                  pl.BlockSpec((128, 128), lambda i: (0, 0))],
        out_specs=pl.BlockSpec((BN_, 128), lambda i: (i, 0)),
        out_shape=jax.ShapeDtypeStruct((N, H), jnp.float32),
    )(x_f, emb_pad)


def _gidx_body(src_ref, attr_ref, o_ref):
    o_ref[...] = attr_ref[...] * N + src_ref[...]


def _gidx(src_r, attr_r):
    return pl.pallas_call(
        _gidx_body,
        grid=(NW,),
        in_specs=[pl.BlockSpec((1, CPT, CHUNK), lambda i: (i, 0, 0)),
                  pl.BlockSpec((1, CPT, CHUNK), lambda i: (i, 0, 0))],
        out_specs=pl.BlockSpec((1, CPT, CHUNK), lambda i: (i, 0, 0)),
        out_shape=jax.ShapeDtypeStruct((NW, CPT, CHUNK), jnp.int32),
    )(src_r, attr_r)


def _comb_body(h_ref, batch_ref, vn_ref, bond_ref, hl_ref, comb_ref):
    iota = lax.broadcasted_iota(jnp.int32, (BN_, G), 1).astype(jnp.float32)
    onehot = (batch_ref[...] == iota).astype(jnp.float32)
    vnb = jnp.dot(onehot, vn_ref[...],
                  preferred_element_type=jnp.float32, precision=HIGH)
    hl = h_ref[...] + vnb
    hl_ref[...] = hl
    for a in range(NUM_EDGE):
        comb_ref[a] = jnp.maximum(hl + bond_ref[a], 0.0)


def _comb(h, batch_f, vn, bond):
    return pl.pallas_call(
        _comb_body,
        grid=(NB,),
        in_specs=[pl.BlockSpec((BN_, 128), lambda i: (i, 0)),
                  pl.BlockSpec((BN_, 1), lambda i: (i, 0)),
                  pl.BlockSpec((G, 128), lambda i: (0, 0)),
                  pl.BlockSpec((NUM_EDGE, 128), lambda i: (0, 0))],
        out_specs=[pl.BlockSpec((BN_, 128), lambda i: (i, 0)),
                   pl.BlockSpec((NUM_EDGE, BN_, 128), lambda i: (0, i, 0))],
        out_shape=[jax.ShapeDtypeStruct((N, H), jnp.float32),
                   jax.ShapeDtypeStruct((NUM_EDGE, N, H), jnp.float32)],
    )(h, batch_f, vn, bond)


def _bn(t, scale, bias):
    m = jnp.mean(t, axis=0)
    d = t - m
    v = jnp.mean(d * d, axis=0)
    return scale * d / jnp.sqrt(v + 1e-5) + bias


def _mlp_body(last, hl_ref, agg_ref, hin_ref, eps_ref, w1_ref, b1_ref,
              s1_ref, c1_ref, w2_ref, b2_ref, s2_ref, c2_ref, batch_ref,
              out_ref, hg_ref=None):
    agg = agg_ref[0, :N, :] + agg_ref[1, :N, :]
    z0 = (1.0 + eps_ref[0, 0]) * hl_ref[...] + agg
    t = jnp.dot(z0, w1_ref[...],
                preferred_element_type=jnp.float32) + b1_ref[...]
    t = jnp.maximum(_bn(t, s1_ref[...], c1_ref[...]), 0.0)
    u = jnp.dot(t, w2_ref[...],
                preferred_element_type=jnp.float32) + b2_ref[...]
    u = _bn(u, s2_ref[...], c2_ref[...])
    if not last:
        u = jnp.maximum(u, 0.0)
    out = u + hin_ref[...]
    out_ref[...] = out
    if last:
        iota = lax.broadcasted_iota(jnp.int32, (G, N), 0).astype(jnp.float32)
        onehot_t = (batch_ref[...] == iota).astype(jnp.float32)
        hg_ref[...] = jnp.dot(onehot_t, out,
                              preferred_element_type=jnp.float32, precision=HIGH)


def _mlp(hl, agg2, h_in, eps, p, batch_row, last):
    full = lambda s: pl.BlockSpec(s, lambda: tuple(0 for _ in s))
    out_shape = [jax.ShapeDtypeStruct((N, H), jnp.float32)]
    out_specs = [full((N, 128))]
    if last:
        out_shape.append(jax.ShapeDtypeStruct((G, H), jnp.float32))
        out_specs.append(full((G, 128)))
    r = pl.pallas_call(
        functools.partial(_mlp_body, last),
        in_specs=[full((N, 128)), full((2, N_PAD, 128)), full((N, 128)),
                  full((1, 1)), full((128, 256)), full((256,)), full((256,)),
                  full((256,)), full((256, 128)), full((128,)), full((128,)),
                  full((128,)), full((1, N))],
        out_specs=out_specs,
        out_shape=out_shape,
    )(hl, agg2, h_in, eps, p['W1'], p['b1'], p['mlp_bn_scale'],
      p['mlp_bn_bias'], p['W2'], p['b2'], p['bn_scale'], p['bn_bias'],
      batch_row)
    return r if last else (r[0], None)


def _vn_body(h_ref, batch_ref, vn_ref, w1_ref, b1_ref, s1_ref, c1_ref,
             w2_ref, b2_ref, s2_ref, c2_ref, out_ref):
    iota = lax.broadcasted_iota(jnp.int32, (G, N), 0).astype(jnp.float32)
    onehot_t = (batch_ref[...] == iota).astype(jnp.float32)
    seg = jnp.dot(onehot_t, h_ref[...],
                  preferred_element_type=jnp.float32, precision=HIGH)
    vt = seg + vn_ref[...]
    t = jnp.dot(vt, w1_ref[...],
                preferred_element_type=jnp.float32) + b1_ref[...]
    t = jnp.maximum(_bn(t, s1_ref[...], c1_ref[...]), 0.0)
    u = jnp.dot(t, w2_ref[...],
                preferred_element_type=jnp.float32) + b2_ref[...]
    u = _bn(u, s2_ref[...], c2_ref[...])
    out_ref[...] = jnp.maximum(u, 0.0)


def _vn_mlp(h, batch_row, vn, q):
    full = lambda s: pl.BlockSpec(s, lambda: tuple(0 for _ in s))
    return pl.pallas_call(
        _vn_body,
        in_specs=[full((N, 128)), full((1, N)), full((G, 128)),
                  full((128, 256)), full((256,)), full((256,)), full((256,)),
                  full((256, 128)), full((128,)), full((128,)), full((128,))],
        out_specs=full((G, 128)),
        out_shape=jax.ShapeDtypeStruct((G, H), jnp.float32),
    )(h, batch_row, vn, q['W1'], q['b1'], q['bn1_scale'], q['bn1_bias'],
      q['W2'], q['b2'], q['bn2_scale'], q['bn2_bias'])


@functools.cache
def _get_sc_edge():
    mesh = plsc.VectorSubcoreMesh(core_axis_name="c", subcore_axis_name="s")

    NSLOT = 2
    HCPT = CPT // 2
    NGRP = HCPT // NSLOT

    @functools.partial(
        pl.kernel,
        out_type=jax.ShapeDtypeStruct((2, N_PAD, H), jnp.float32),
        mesh=mesh,
        scratch_types=[
            pltpu.VMEM((HCPT, CHUNK), jnp.int32),
            pltpu.VMEM((HCPT, CHUNK), jnp.int32),
            pltpu.VMEM((NSLOT, CHUNK, H), jnp.float32),
            pltpu.VMEM_SHARED((N_PAD, H), jnp.float32),
            [pltpu.SemaphoreType.DMA] * NSLOT,
            [pltpu.SemaphoreType.DMA] * NSLOT,
        ],
    )
    def _sc_edge_kernel(comb_hbm, gidx_hbm, didx_hbm, zeros_hbm, out_hbm,
                        gidx_v, didx_v, rows_v, agg_sh, gsem, ssem):
        c = lax.axis_index("c")
        s = lax.axis_index("s")
        wid = s * 2 + c

        pltpu.sync_copy(zeros_hbm,
                        agg_sh.at[pl.ds(s * ROWS_PER_TILE, ROWS_PER_TILE)])
        plsc.subcore_barrier()

        def wait_slot(b, sem):
            pltpu.make_async_copy(comb_hbm.at[pl.ds(0, CHUNK)],
                                  rows_v.at[b], sem[b]).wait()

        for half in range(2):
            pltpu.sync_copy(gidx_hbm.at[wid, pl.ds(half * HCPT, HCPT)], gidx_v)
            pltpu.sync_copy(didx_hbm.at[wid, pl.ds(half * HCPT, HCPT)], didx_v)

            for b in range(NSLOT):
                pltpu.async_copy(comb_hbm.at[gidx_v.at[b]], rows_v.at[b],
                                 gsem[b])

            @pl.loop(0, NGRP - 1)
            def _(g):
                for b in range(NSLOT):
                    wait_slot(b, gsem)
                    pltpu.async_copy(rows_v.at[b],
                                     agg_sh.at[didx_v.at[g * NSLOT + b]],
                                     ssem[b], add=True)
                for b in range(NSLOT):
                    wait_slot(b, ssem)
                    pltpu.async_copy(
                        comb_hbm.at[gidx_v.at[(g + 1) * NSLOT + b]],
                        rows_v.at[b], gsem[b])

            for b in range(NSLOT):
                wait_slot(b, gsem)
                pltpu.async_copy(rows_v.at[b],
                                 agg_sh.at[didx_v.at[HCPT - NSLOT + b]],
                                 ssem[b], add=True)
            for b in range(NSLOT):
                wait_slot(b, ssem)

        plsc.subcore_barrier()
        pltpu.sync_copy(agg_sh.at[pl.ds(s * ROWS_PER_TILE, ROWS_PER_TILE)],
                        out_hbm.at[c, pl.ds(s * ROWS_PER_TILE, ROWS_PER_TILE)])

    return _sc_edge_kernel


def _sc_edge(comb, gidx, didx, zeros_hbm):
    return _get_sc_edge()(comb, gidx, didx, zeros_hbm)


def kernel(x, edge_index, edge_attr, batch, params):
    x_f = x.astype(jnp.float32).reshape(N, 1)
    batch_f = batch.astype(jnp.float32).reshape(N, 1)
    batch_row = batch.astype(jnp.float32).reshape(1, N)

    src = edge_index[0].astype(jnp.int32)
    dst = edge_index[1].astype(jnp.int32)
    attr = edge_attr.astype(jnp.int32)
    npad = E_PAD - E
    pad_i = jnp.arange(npad, dtype=jnp.int32)
    src_r = jnp.concatenate([src, pad_i % N]).reshape(NW, CPT, CHUNK)
    attr_r = jnp.concatenate([attr, jnp.zeros((npad,), jnp.int32)]
                             ).reshape(NW, CPT, CHUNK)
    didx = jnp.concatenate([dst, N + pad_i % (N_PAD - N)]
                           ).reshape(NW, CPT, CHUNK)
    gidx = _gidx(src_r, attr_r)

    emb_pad = jnp.zeros((128, 128), jnp.float32).at[:NUM_ATOM].set(
        params['atom_emb'])
    zeros_hbm = jnp.zeros((ROWS_PER_TILE, H), jnp.float32)

    h = _h0(x_f, emb_pad)
    vn = jnp.broadcast_to(params['vn_emb'], (G, H))

    hg = None
    for l in range(L):
        p = params['layers'][l]
        hl, comb = _comb(h, batch_f, vn, p['bond_emb'])
        agg2 = _sc_edge(comb.reshape(NUM_EDGE * N, H), gidx, didx, zeros_hbm)
        if l < L - 1:
            vn = _vn_mlp(h, batch_row, vn, params['vn_mlps'][l])
        eps = p['eps'].reshape(1, 1)
        h, hg = _mlp(hl, agg2, h, eps, p, batch_row, last=(l == L - 1))

    return hg, h


# speedup vs baseline: 9.0443x; 1.0036x over previous
import functools

import jax
import jax.numpy as jnp
from jax import lax
from jax.experimental import pallas as pl
from jax.experimental.pallas import tpu as pltpu
from jax.experimental.pallas import tpu_sc as plsc

N = 10000
E = 320000
H = 128
L = 3
G = 64
NUM_ATOM = 119
NUM_EDGE = 5

NW = 32
CHUNK = 128
CPT = 80
E_PAD = NW * CPT * CHUNK
N_PAD = 10240
ROWS_PER_TILE = N_PAD // 16
NB = 10
BN_ = N // NB
HIGH = lax.Precision.HIGHEST


def _h0_body(x_ref, emb_ref, o_ref):
    iota = lax.broadcasted_iota(jnp.int32, (BN_, 128), 1).astype(jnp.float32)
    onehot = (x_ref[...] == iota).astype(jnp.float32)
    o_ref[...] = jnp.dot(onehot, emb_ref[...],
                         preferred_element_type=jnp.float32, precision=HIGH)


_PAR = pltpu.CompilerParams(dimension_semantics=("parallel",))


def _h0(x_f, emb_pad):
    return pl.pallas_call(
        _h0_body,
        grid=(NB,),
        compiler_params=_PAR,
        in_specs=[pl.BlockSpec((BN_, 1), lambda i: (i, 0)),
                  pl.BlockSpec((128, 128), lambda i: (0, 0))],
        out_specs=pl.BlockSpec((BN_, 128), lambda i: (i, 0)),
        out_shape=jax.ShapeDtypeStruct((N, H), jnp.float32),
    )(x_f, emb_pad)


def _gidx_body(src_ref, attr_ref, o_ref):
    o_ref[...] = attr_ref[...] * N + src_ref[...]


def _gidx(src_r, attr_r):
    return pl.pallas_call(
        _gidx_body,
        grid=(NW,),
        compiler_params=_PAR,
        in_specs=[pl.BlockSpec((1, CPT, CHUNK), lambda i: (i, 0, 0)),
                  pl.BlockSpec((1, CPT, CHUNK), lambda i: (i, 0, 0))],
        out_specs=pl.BlockSpec((1, CPT, CHUNK), lambda i: (i, 0, 0)),
        out_shape=jax.ShapeDtypeStruct((NW, CPT, CHUNK), jnp.int32),
    )(src_r, attr_r)


def _comb_body(h_ref, batch_ref, vn_ref, bond_ref, hl_ref, comb_ref):
    iota = lax.broadcasted_iota(jnp.int32, (BN_, G), 1).astype(jnp.float32)
    onehot = (batch_ref[...] == iota).astype(jnp.float32)
    vnb = jnp.dot(onehot, vn_ref[...],
                  preferred_element_type=jnp.float32, precision=HIGH)
    hl = h_ref[...] + vnb
    hl_ref[...] = hl
    for a in range(NUM_EDGE):
        comb_ref[a] = jnp.maximum(hl + bond_ref[a], 0.0)


def _comb(h, batch_f, vn, bond):
    return pl.pallas_call(
        _comb_body,
        grid=(NB,),
        compiler_params=_PAR,
        in_specs=[pl.BlockSpec((BN_, 128), lambda i: (i, 0)),
                  pl.BlockSpec((BN_, 1), lambda i: (i, 0)),
                  pl.BlockSpec((G, 128), lambda i: (0, 0)),
                  pl.BlockSpec((NUM_EDGE, 128), lambda i: (0, 0))],
        out_specs=[pl.BlockSpec((BN_, 128), lambda i: (i, 0)),
                   pl.BlockSpec((NUM_EDGE, BN_, 128), lambda i: (0, i, 0))],
        out_shape=[jax.ShapeDtypeStruct((N, H), jnp.float32),
                   jax.ShapeDtypeStruct((NUM_EDGE, N, H), jnp.float32)],
    )(h, batch_f, vn, bond)


def _bn(t, scale, bias):
    m = jnp.mean(t, axis=0)
    d = t - m
    v = jnp.mean(d * d, axis=0)
    return scale * d / jnp.sqrt(v + 1e-5) + bias


def _mlp_body(last, hl_ref, agg_ref, hin_ref, eps_ref, w1_ref, b1_ref,
              s1_ref, c1_ref, w2_ref, b2_ref, s2_ref, c2_ref, batch_ref,
              out_ref, hg_ref=None):
    agg = agg_ref[0, :N, :] + agg_ref[1, :N, :]
    z0 = (1.0 + eps_ref[0, 0]) * hl_ref[...] + agg
    t = jnp.dot(z0, w1_ref[...],
                preferred_element_type=jnp.float32) + b1_ref[...]
    t = jnp.maximum(_bn(t, s1_ref[...], c1_ref[...]), 0.0)
    u = jnp.dot(t, w2_ref[...],
                preferred_element_type=jnp.float32) + b2_ref[...]
    u = _bn(u, s2_ref[...], c2_ref[...])
    if not last:
        u = jnp.maximum(u, 0.0)
    out = u + hin_ref[...]
    out_ref[...] = out
    if last:
        iota = lax.broadcasted_iota(jnp.int32, (G, N), 0).astype(jnp.float32)
        onehot_t = (batch_ref[...] == iota).astype(jnp.float32)
        hg_ref[...] = jnp.dot(onehot_t, out,
                              preferred_element_type=jnp.float32, precision=HIGH)


def _mlp(hl, agg2, h_in, eps, p, batch_row, last):
    full = lambda s: pl.BlockSpec(s, lambda: tuple(0 for _ in s))
    out_shape = [jax.ShapeDtypeStruct((N, H), jnp.float32)]
    out_specs = [full((N, 128))]
    if last:
        out_shape.append(jax.ShapeDtypeStruct((G, H), jnp.float32))
        out_specs.append(full((G, 128)))
    r = pl.pallas_call(
        functools.partial(_mlp_body, last),
        in_specs=[full((N, 128)), full((2, N_PAD, 128)), full((N, 128)),
                  full((1, 1)), full((128, 256)), full((256,)), full((256,)),
                  full((256,)), full((256, 128)), full((128,)), full((128,)),
                  full((128,)), full((1, N))],
        out_specs=out_specs,
        out_shape=out_shape,
    )(hl, agg2, h_in, eps, p['W1'], p['b1'], p['mlp_bn_scale'],
      p['mlp_bn_bias'], p['W2'], p['b2'], p['bn_scale'], p['bn_bias'],
      batch_row)
    return r if last else (r[0], None)


def _vn_body(h_ref, batch_ref, vn_ref, w1_ref, b1_ref, s1_ref, c1_ref,
             w2_ref, b2_ref, s2_ref, c2_ref, out_ref):
    iota = lax.broadcasted_iota(jnp.int32, (G, N), 0).astype(jnp.float32)
    onehot_t = (batch_ref[...] == iota).astype(jnp.float32)
    seg = jnp.dot(onehot_t, h_ref[...],
                  preferred_element_type=jnp.float32, precision=HIGH)
    vt = seg + vn_ref[...]
    t = jnp.dot(vt, w1_ref[...],
                preferred_element_type=jnp.float32) + b1_ref[...]
    t = jnp.maximum(_bn(t, s1_ref[...], c1_ref[...]), 0.0)
    u = jnp.dot(t, w2_ref[...],
                preferred_element_type=jnp.float32) + b2_ref[...]
    u = _bn(u, s2_ref[...], c2_ref[...])
    out_ref[...] = jnp.maximum(u, 0.0)


def _vn_mlp(h, batch_row, vn, q):
    full = lambda s: pl.BlockSpec(s, lambda: tuple(0 for _ in s))
    return pl.pallas_call(
        _vn_body,
        in_specs=[full((N, 128)), full((1, N)), full((G, 128)),
                  full((128, 256)), full((256,)), full((256,)), full((256,)),
                  full((256, 128)), full((128,)), full((128,)), full((128,))],
        out_specs=full((G, 128)),
        out_shape=jax.ShapeDtypeStruct((G, H), jnp.float32),
    )(h, batch_row, vn, q['W1'], q['b1'], q['bn1_scale'], q['bn1_bias'],
      q['W2'], q['b2'], q['bn2_scale'], q['bn2_bias'])


@functools.cache
def _get_sc_edge():
    mesh = plsc.VectorSubcoreMesh(core_axis_name="c", subcore_axis_name="s")

    NSLOT = 2
    HCPT = CPT // 2
    NGRP = HCPT // NSLOT

    @functools.partial(
        pl.kernel,
        out_type=jax.ShapeDtypeStruct((2, N_PAD, H), jnp.float32),
        mesh=mesh,
        scratch_types=[
            pltpu.VMEM((HCPT, CHUNK), jnp.int32),
            pltpu.VMEM((HCPT, CHUNK), jnp.int32),
            pltpu.VMEM((NSLOT, CHUNK, H), jnp.float32),
            pltpu.VMEM_SHARED((N_PAD, H), jnp.float32),
            [pltpu.SemaphoreType.DMA] * NSLOT,
            [pltpu.SemaphoreType.DMA] * NSLOT,
        ],
    )
    def _sc_edge_kernel(comb_hbm, gidx_hbm, didx_hbm, zeros_hbm, out_hbm,
                        gidx_v, didx_v, rows_v, agg_sh, gsem, ssem):
        c = lax.axis_index("c")
        s = lax.axis_index("s")
        wid = s * 2 + c

        pltpu.sync_copy(zeros_hbm,
                        agg_sh.at[pl.ds(s * ROWS_PER_TILE, ROWS_PER_TILE)])
        plsc.subcore_barrier()

        def wait_slot(b, sem):
            pltpu.make_async_copy(comb_hbm.at[pl.ds(0, CHUNK)],
                                  rows_v.at[b], sem[b]).wait()

        for half in range(2):
            pltpu.sync_copy(gidx_hbm.at[wid, pl.ds(half * HCPT, HCPT)], gidx_v)
            pltpu.sync_copy(didx_hbm.at[wid, pl.ds(half * HCPT, HCPT)], didx_v)

            for b in range(NSLOT):
                pltpu.async_copy(comb_hbm.at[gidx_v.at[b]], rows_v.at[b],
                                 gsem[b])

            @pl.loop(0, NGRP - 1)
            def _(g):
                for b in range(NSLOT):
                    wait_slot(b, gsem)
                    pltpu.async_copy(rows_v.at[b],
                                     agg_sh.at[didx_v.at[g * NSLOT + b]],
                                     ssem[b], add=True)
                for b in range(NSLOT):
                    wait_slot(b, ssem)
                    pltpu.async_copy(
                        comb_hbm.at[gidx_v.at[(g + 1) * NSLOT + b]],
                        rows_v.at[b], gsem[b])

            for b in range(NSLOT):
                wait_slot(b, gsem)
                pltpu.async_copy(rows_v.at[b],
                                 agg_sh.at[didx_v.at[HCPT - NSLOT + b]],
                                 ssem[b], add=True)
            for b in range(NSLOT):
                wait_slot(b, ssem)

        plsc.subcore_barrier()
        pltpu.sync_copy(agg_sh.at[pl.ds(s * ROWS_PER_TILE, ROWS_PER_TILE)],
                        out_hbm.at[c, pl.ds(s * ROWS_PER_TILE, ROWS_PER_TILE)])

    return _sc_edge_kernel


def _sc_edge(comb, gidx, didx, zeros_hbm):
    return _get_sc_edge()(comb, gidx, didx, zeros_hbm)


def kernel(x, edge_index, edge_attr, batch, params):
    x_f = x.astype(jnp.float32).reshape(N, 1)
    batch_f = batch.astype(jnp.float32).reshape(N, 1)
    batch_row = batch.astype(jnp.float32).reshape(1, N)

    src = edge_index[0].astype(jnp.int32)
    dst = edge_index[1].astype(jnp.int32)
    attr = edge_attr.astype(jnp.int32)
    npad = E_PAD - E
    pad_i = jnp.arange(npad, dtype=jnp.int32)
    src_r = jnp.concatenate([src, pad_i % N]).reshape(NW, CPT, CHUNK)
    attr_r = jnp.concatenate([attr, jnp.zeros((npad,), jnp.int32)]
                             ).reshape(NW, CPT, CHUNK)
    didx = jnp.concatenate([dst, N + pad_i % (N_PAD - N)]
                           ).reshape(NW, CPT, CHUNK)
    gidx = _gidx(src_r, attr_r)

    emb_pad = jnp.zeros((128, 128), jnp.float32).at[:NUM_ATOM].set(
        params['atom_emb'])
    zeros_hbm = jnp.zeros((ROWS_PER_TILE, H), jnp.float32)

    h = _h0(x_f, emb_pad)
    vn = jnp.broadcast_to(params['vn_emb'], (G, H))

    hg = None
    for l in range(L):
        p = params['layers'][l]
        hl, comb = _comb(h, batch_f, vn, p['bond_emb'])
        agg2 = _sc_edge(comb.reshape(NUM_EDGE * N, H), gidx, didx, zeros_hbm)
        if l < L - 1:
            vn = _vn_mlp(h, batch_row, vn, params['vn_mlps'][l])
        eps = p['eps'].reshape(1, 1)
        h, hg = _mlp(hl, agg2, h, eps, p, batch_row, last=(l == L - 1))

    return hg, h


# speedup vs baseline: 9.4802x; 1.0482x over previous
import functools

import jax
import jax.numpy as jnp
from jax import lax
from jax.experimental import pallas as pl
from jax.experimental.pallas import tpu as pltpu
from jax.experimental.pallas import tpu_sc as plsc

N = 10000
E = 320000
H = 128
L = 3
G = 64
NUM_ATOM = 119
NUM_EDGE = 5

NW = 32
CHUNK = 128
CPT = 80
E_PAD = NW * CPT * CHUNK
N_PAD = 10240
ROWS_PER_TILE = N_PAD // 16
NB = 10
BN_ = N // NB
HIGH = lax.Precision.HIGHEST


_PAR = pltpu.CompilerParams(dimension_semantics=("parallel",))


def _comb0_body(x_ref, emb_ref, batch_ref, vn_ref, bond_ref,
                h0_ref, hl_ref, comb_ref):
    iota128 = lax.broadcasted_iota(jnp.int32, (BN_, 128), 1).astype(jnp.float32)
    onehot_x = (x_ref[...] == iota128).astype(jnp.float32)
    h0 = jnp.dot(onehot_x, emb_ref[...],
                 preferred_element_type=jnp.float32, precision=HIGH)
    h0_ref[...] = h0
    iota = lax.broadcasted_iota(jnp.int32, (BN_, G), 1).astype(jnp.float32)
    onehot = (batch_ref[...] == iota).astype(jnp.float32)
    vnb = jnp.dot(onehot, vn_ref[...],
                  preferred_element_type=jnp.float32, precision=HIGH)
    hl = h0 + vnb
    hl_ref[...] = hl
    for a in range(NUM_EDGE):
        comb_ref[a] = jnp.maximum(hl + bond_ref[a], 0.0)


def _comb0(x_f, emb_pad, batch_f, vn, bond):
    return pl.pallas_call(
        _comb0_body,
        grid=(NB,),
        compiler_params=_PAR,
        in_specs=[pl.BlockSpec((BN_, 1), lambda i: (i, 0)),
                  pl.BlockSpec((128, 128), lambda i: (0, 0)),
                  pl.BlockSpec((BN_, 1), lambda i: (i, 0)),
                  pl.BlockSpec((G, 128), lambda i: (0, 0)),
                  pl.BlockSpec((NUM_EDGE, 128), lambda i: (0, 0))],
        out_specs=[pl.BlockSpec((BN_, 128), lambda i: (i, 0)),
                   pl.BlockSpec((BN_, 128), lambda i: (i, 0)),
                   pl.BlockSpec((NUM_EDGE, BN_, 128), lambda i: (0, i, 0))],
        out_shape=[jax.ShapeDtypeStruct((N, H), jnp.float32),
                   jax.ShapeDtypeStruct((N, H), jnp.float32),
                   jax.ShapeDtypeStruct((NUM_EDGE, N, H), jnp.float32)],
    )(x_f, emb_pad, batch_f, vn, bond)


def _comb_body(h_ref, batch_ref, vn_ref, bond_ref, hl_ref, comb_ref):
    iota = lax.broadcasted_iota(jnp.int32, (BN_, G), 1).astype(jnp.float32)
    onehot = (batch_ref[...] == iota).astype(jnp.float32)
    vnb = jnp.dot(onehot, vn_ref[...],
                  preferred_element_type=jnp.float32, precision=HIGH)
    hl = h_ref[...] + vnb
    hl_ref[...] = hl
    for a in range(NUM_EDGE):
        comb_ref[a] = jnp.maximum(hl + bond_ref[a], 0.0)


def _comb(h, batch_f, vn, bond):
    return pl.pallas_call(
        _comb_body,
        grid=(NB,),
        compiler_params=_PAR,
        in_specs=[pl.BlockSpec((BN_, 128), lambda i: (i, 0)),
                  pl.BlockSpec((BN_, 1), lambda i: (i, 0)),
                  pl.BlockSpec((G, 128), lambda i: (0, 0)),
                  pl.BlockSpec((NUM_EDGE, 128), lambda i: (0, 0))],
        out_specs=[pl.BlockSpec((BN_, 128), lambda i: (i, 0)),
                   pl.BlockSpec((NUM_EDGE, BN_, 128), lambda i: (0, i, 0))],
        out_shape=[jax.ShapeDtypeStruct((N, H), jnp.float32),
                   jax.ShapeDtypeStruct((NUM_EDGE, N, H), jnp.float32)],
    )(h, batch_f, vn, bond)


def _bn(t, scale, bias):
    m = jnp.mean(t, axis=0)
    d = t - m
    v = jnp.mean(d * d, axis=0)
    return scale * d / jnp.sqrt(v + 1e-5) + bias


def _mlp_body(last, hl_ref, agg_ref, hin_ref, eps_ref, w1_ref, b1_ref,
              s1_ref, c1_ref, w2_ref, b2_ref, s2_ref, c2_ref, batch_ref,
              out_ref, hg_ref=None):
    agg = agg_ref[0, :N, :] + agg_ref[1, :N, :]
    z0 = (1.0 + eps_ref[0, 0]) * hl_ref[...] + agg
    t = jnp.dot(z0, w1_ref[...],
                preferred_element_type=jnp.float32) + b1_ref[...]
    t = jnp.maximum(_bn(t, s1_ref[...], c1_ref[...]), 0.0)
    u = jnp.dot(t, w2_ref[...],
                preferred_element_type=jnp.float32) + b2_ref[...]
    u = _bn(u, s2_ref[...], c2_ref[...])
    if not last:
        u = jnp.maximum(u, 0.0)
    out = u + hin_ref[...]
    out_ref[...] = out
    if last:
        iota = lax.broadcasted_iota(jnp.int32, (G, N), 0).astype(jnp.float32)
        onehot_t = (batch_ref[...] == iota).astype(jnp.float32)
        hg_ref[...] = jnp.dot(onehot_t, out,
                              preferred_element_type=jnp.float32, precision=HIGH)


def _mlp(hl, agg2, h_in, eps, p, batch_row, last):
    full = lambda s: pl.BlockSpec(s, lambda: tuple(0 for _ in s))
    out_shape = [jax.ShapeDtypeStruct((N, H), jnp.float32)]
    out_specs = [full((N, 128))]
    if last:
        out_shape.append(jax.ShapeDtypeStruct((G, H), jnp.float32))
        out_specs.append(full((G, 128)))
    r = pl.pallas_call(
        functools.partial(_mlp_body, last),
        in_specs=[full((N, 128)), full((2, N_PAD, 128)), full((N, 128)),
                  full((1, 1)), full((128, 256)), full((256,)), full((256,)),
                  full((256,)), full((256, 128)), full((128,)), full((128,)),
                  full((128,)), full((1, N))],
        out_specs=out_specs,
        out_shape=out_shape,
    )(hl, agg2, h_in, eps, p['W1'], p['b1'], p['mlp_bn_scale'],
      p['mlp_bn_bias'], p['W2'], p['b2'], p['bn_scale'], p['bn_bias'],
      batch_row)
    return r if last else (r[0], None)


def _vn_body(h_ref, batch_ref, vn_ref, w1_ref, b1_ref, s1_ref, c1_ref,
             w2_ref, b2_ref, s2_ref, c2_ref, out_ref):
    iota = lax.broadcasted_iota(jnp.int32, (G, N), 0).astype(jnp.float32)
    onehot_t = (batch_ref[...] == iota).astype(jnp.float32)
    seg = jnp.dot(onehot_t, h_ref[...],
                  preferred_element_type=jnp.float32, precision=HIGH)
    vt = seg + vn_ref[...]
    t = jnp.dot(vt, w1_ref[...],
                preferred_element_type=jnp.float32) + b1_ref[...]
    t = jnp.maximum(_bn(t, s1_ref[...], c1_ref[...]), 0.0)
    u = jnp.dot(t, w2_ref[...],
                preferred_element_type=jnp.float32) + b2_ref[...]
    u = _bn(u, s2_ref[...], c2_ref[...])
    out_ref[...] = jnp.maximum(u, 0.0)


def _vn_mlp(h, batch_row, vn, q):
    full = lambda s: pl.BlockSpec(s, lambda: tuple(0 for _ in s))
    return pl.pallas_call(
        _vn_body,
        in_specs=[full((N, 128)), full((1, N)), full((G, 128)),
                  full((128, 256)), full((256,)), full((256,)), full((256,)),
                  full((256, 128)), full((128,)), full((128,)), full((128,))],
        out_specs=full((G, 128)),
        out_shape=jax.ShapeDtypeStruct((G, H), jnp.float32),
    )(h, batch_row, vn, q['W1'], q['b1'], q['bn1_scale'], q['bn1_bias'],
      q['W2'], q['b2'], q['bn2_scale'], q['bn2_bias'])


@functools.cache
def _get_sc_edge():
    mesh = plsc.VectorSubcoreMesh(core_axis_name="c", subcore_axis_name="s")

    NSLOT = 2
    HCPT = CPT // 2

    @functools.partial(
        pl.kernel,
        out_type=jax.ShapeDtypeStruct((2, N_PAD, H), jnp.float32),
        mesh=mesh,
        scratch_types=[
            pltpu.VMEM((HCPT, CHUNK), jnp.int32),
            pltpu.VMEM((HCPT, CHUNK), jnp.int32),
            pltpu.VMEM((HCPT, CHUNK), jnp.int32),
            pltpu.VMEM((NSLOT, CHUNK, H), jnp.float32),
            pltpu.VMEM_SHARED((N_PAD, H), jnp.float32),
            [pltpu.SemaphoreType.DMA] * NSLOT,
            [pltpu.SemaphoreType.DMA] * NSLOT,
        ],
    )
    def _sc_edge_kernel(comb_hbm, src_hbm, attr_hbm, dst_hbm, out_hbm,
                        src_v, attr_v, didx_v, rows_v, agg_sh, gsem, ssem):
        c = lax.axis_index("c")
        s = lax.axis_index("s")
        wid = s * 2 + c

        @pl.loop(0, CHUNK)
        def _(r):
            for k in range(H // 16):
                rows_v[0, r, pl.ds(k * 16, 16)] = jnp.zeros((16,), jnp.float32)

        for k in range(ROWS_PER_TILE // CHUNK):
            pltpu.sync_copy(
                rows_v.at[0],
                agg_sh.at[pl.ds(s * ROWS_PER_TILE + k * CHUNK, CHUNK)])
        plsc.subcore_barrier()

        def wait_slot(b, sem):
            pltpu.make_async_copy(comb_hbm.at[pl.ds(0, CHUNK)],
                                  rows_v.at[b], sem[b]).wait()

        for half in range(2):
            hsl = pl.ds(half * HCPT, HCPT)
            pltpu.sync_copy(src_hbm.at[wid, hsl], src_v)
            pltpu.sync_copy(attr_hbm.at[wid, hsl], attr_v)
            pltpu.sync_copy(dst_hbm.at[wid, hsl], didx_v)

            @pl.loop(0, HCPT)
            def _(r):
                for k in range(CHUNK // 16):
                    sl = pl.ds(k * 16, 16)
                    src_v[r, sl] = attr_v[r, sl] * N + src_v[r, sl]

            for b in range(NSLOT):
                pltpu.async_copy(comb_hbm.at[src_v.at[b]], rows_v.at[b],
                                 gsem[b])

            @pl.loop(0, HCPT // NSLOT - 1)
            def _(g):
                for b in range(NSLOT):
                    wait_slot(b, gsem)
                    pltpu.async_copy(rows_v.at[b],
                                     agg_sh.at[didx_v.at[g * NSLOT + b]],
                                     ssem[b], add=True)
                for b in range(NSLOT):
                    wait_slot(b, ssem)
                    pltpu.async_copy(
                        comb_hbm.at[src_v.at[(g + 1) * NSLOT + b]],
                        rows_v.at[b], gsem[b])

            for b in range(NSLOT):
                wait_slot(b, gsem)
                pltpu.async_copy(rows_v.at[b],
                                 agg_sh.at[didx_v.at[HCPT - NSLOT + b]],
                                 ssem[b], add=True)
            for b in range(NSLOT):
                wait_slot(b, ssem)

        plsc.subcore_barrier()
        pltpu.sync_copy(agg_sh.at[pl.ds(s * ROWS_PER_TILE, ROWS_PER_TILE)],
                        out_hbm.at[c, pl.ds(s * ROWS_PER_TILE, ROWS_PER_TILE)])

    return _sc_edge_kernel


def _sc_edge(comb, srcr, attrr, dstr):
    return _get_sc_edge()(comb, srcr, attrr, dstr)


def kernel(x, edge_index, edge_attr, batch, params):
    x_f = x.astype(jnp.float32).reshape(N, 1)
    batch_f = batch.astype(jnp.float32).reshape(N, 1)
    batch_row = batch.astype(jnp.float32).reshape(1, N)

    src = edge_index[0].astype(jnp.int32)
    dst = edge_index[1].astype(jnp.int32)
    attr = edge_attr.astype(jnp.int32)
    npad = E_PAD - E
    pad_i = jnp.arange(npad, dtype=jnp.int32)
    srcr = jnp.concatenate([src, pad_i % N]).reshape(NW, CPT, CHUNK)
    attrr = jnp.concatenate([attr, jnp.zeros((npad,), jnp.int32)]
                            ).reshape(NW, CPT, CHUNK)
    dstr = jnp.concatenate([dst, N + pad_i % (N_PAD - N)]
                           ).reshape(NW, CPT, CHUNK)

    emb_pad = jnp.zeros((128, 128), jnp.float32).at[:NUM_ATOM].set(
        params['atom_emb'])
    vn = jnp.broadcast_to(params['vn_emb'], (G, H))

    hg = None
    h = None
    for l in range(L):
        p = params['layers'][l]
        if l == 0:
            h, hl, comb = _comb0(x_f, emb_pad, batch_f, vn, p['bond_emb'])
        else:
            hl, comb = _comb(h, batch_f, vn, p['bond_emb'])
        agg2 = _sc_edge(comb.reshape(NUM_EDGE * N, H), srcr, attrr, dstr)
        if l < L - 1:
            vn = _vn_mlp(h, batch_row, vn, params['vn_mlps'][l])
        eps = p['eps'].reshape(1, 1)
        h, hg = _mlp(hl, agg2, h, eps, p, batch_row, last=(l == L - 1))

    return hg, h
